# Initial kernel scaffold; baseline (speedup 1.0000x reference)
import jax
import jax.numpy as jnp
from jax.experimental import pallas as pl


def kernel(x, theta, gumbel_u, accept_u):
    raise NotImplementedError("write your pallas kernel here")


# speedup vs baseline: 3.1739x; 3.1739x over previous
import jax
import jax.numpy as jnp
from jax import lax
from jax.experimental import pallas as pl

B, D, V = 64, 32, 8192
TEMP = 2.0


def _step_kernel(x_ref, th_ref, gu_ref, au_ref, out_ref):
    xb = x_ref[0]
    gu = gu_ref[0]
    th = th_ref[...]

    iota_v = lax.broadcasted_iota(jnp.int32, (D, V), 1)
    iota_d = lax.broadcasted_iota(jnp.int32, (D, 1), 0)

    c = jnp.sum(th * xb, axis=1, keepdims=True)

    fd = (th - c) / TEMP
    fl = fd - 1e9 * xb
    g = -jnp.log(-jnp.log(gu))
    score = fl + g

    rowmax = jnp.max(score, axis=1, keepdims=True)
    rowarg = jnp.min(jnp.where(score == rowmax, iota_v, V),
                     axis=1, keepdims=True)
    gmax = jnp.max(rowmax, axis=0, keepdims=True)
    is_gmax = rowmax == gmax
    dstar = jnp.min(jnp.where(is_gmax, iota_d, D), axis=0, keepdims=True)
    is_dstar = iota_d == dstar
    vstar = jnp.sum(jnp.where(is_dstar, rowarg, 0), axis=0, keepdims=True)
    new_mask = is_dstar & (iota_v == vstar)

    m1 = jnp.max(fl, axis=1, keepdims=True)
    m1 = jnp.max(m1, axis=0, keepdims=True)
    z1 = jnp.sum(jnp.exp(fl - m1), axis=1, keepdims=True)
    z1 = jnp.sum(z1, axis=0, keepdims=True)
    lse_f = m1 + jnp.log(z1)
    fl_new = jnp.sum(jnp.where(new_mask, fl, 0.0), axis=1, keepdims=True)
    fl_new = jnp.sum(fl_new, axis=0, keepdims=True)
    lp_forward = fl_new - lse_f

    newrow = new_mask.astype(xb.dtype)
    xd = jnp.where(is_dstar, newrow, xb)
    th_new = jnp.sum(th * newrow, axis=1, keepdims=True)
    th_new = jnp.sum(th_new, axis=0, keepdims=True)
    c_prime = jnp.where(is_dstar, th_new, c)

    rl = (th - c_prime) / TEMP - 1e9 * xd
    m2 = jnp.max(rl, axis=1, keepdims=True)
    m2 = jnp.max(m2, axis=0, keepdims=True)
    z2 = jnp.sum(jnp.exp(rl - m2), axis=1, keepdims=True)
    z2 = jnp.sum(z2, axis=0, keepdims=True)
    lse_r = m2 + jnp.log(z2)
    old_mask = is_dstar & (xb > 0.5)
    rl_old = jnp.sum(jnp.where(old_mask, rl, 0.0), axis=1, keepdims=True)
    rl_old = jnp.sum(rl_old, axis=0, keepdims=True)
    lp_reverse = rl_old - lse_r

    e_new = jnp.sum(xd * th, axis=1, keepdims=True)
    e_new = jnp.sum(e_new, axis=0, keepdims=True)
    e_old = jnp.sum(c, axis=0, keepdims=True)
    la = (e_new - e_old) + lp_reverse - lp_forward
    accept = jnp.exp(la) > au_ref[0]

    take_new = accept & is_dstar
    out_ref[0] = jnp.where(take_new, newrow, xb)


@jax.jit
def kernel(x, theta, gumbel_u, accept_u):
    th = theta.reshape(D, V)
    gu = gumbel_u.reshape(B, D, V)
    au = accept_u.reshape(B, 1, 1)
    return pl.pallas_call(
        _step_kernel,
        grid=(B,),
        in_specs=[
            pl.BlockSpec((1, D, V), lambda b: (b, 0, 0)),
            pl.BlockSpec((D, V), lambda b: (0, 0)),
            pl.BlockSpec((1, D, V), lambda b: (b, 0, 0)),
            pl.BlockSpec((1, 1, 1), lambda b: (b, 0, 0)),
        ],
        out_specs=pl.BlockSpec((1, D, V), lambda b: (b, 0, 0)),
        out_shape=jax.ShapeDtypeStruct((B, D, V), x.dtype),
    )(x, th, gu, au)


# speedup vs baseline: 3.9602x; 1.2478x over previous
import jax
import jax.numpy as jnp
from jax import lax
from jax.experimental import pallas as pl

B, D, V = 64, 32, 8192
TEMP = 2.0


def _step_kernel(x_ref, th_ref, gu_ref, au_ref, out_ref):
    xb = x_ref[0]
    gu = gu_ref[0]
    th = th_ref[...]

    iota_v = lax.broadcasted_iota(jnp.int32, (D, V), 1)
    iota_d = lax.broadcasted_iota(jnp.int32, (D, 1), 0)

    c = jnp.sum(th * xb, axis=1, keepdims=True)
    fd = (th - c) / TEMP
    P = jnp.exp(fd)
    w = -jnp.log(gu)

    sc = jnp.where(xb > 0.5, 0.0, P) / w
    rowmax = jnp.max(sc, axis=1, keepdims=True)
    rowarg = jnp.min(jnp.where(sc == rowmax, iota_v, V),
                     axis=1, keepdims=True)
    gmax = jnp.max(rowmax, axis=0, keepdims=True)
    dstar = jnp.min(jnp.where(rowmax == gmax, iota_d, D),
                    axis=0, keepdims=True)
    is_dstar = iota_d == dstar
    vstar = jnp.sum(jnp.where(is_dstar, rowarg, 0), axis=0, keepdims=True)
    new_mask = is_dstar & (iota_v == vstar)
    newrow = new_mask.astype(xb.dtype)

    rowP = jnp.sum(P, axis=1, keepdims=True)
    rowz = rowP - 1.0
    z1 = jnp.sum(rowz, axis=0, keepdims=True)
    lse_f = jnp.log(z1)

    th_old = jnp.sum(jnp.where(is_dstar, c, 0.0), axis=0, keepdims=True)
    th_new = jnp.sum(th * newrow, axis=1, keepdims=True)
    th_new = jnp.sum(th_new, axis=0, keepdims=True)
    delta = (th_old - th_new) / TEMP
    lp_forward = -delta - lse_f

    rowz_dstar = jnp.sum(jnp.where(is_dstar, rowz, 0.0), axis=0,
                         keepdims=True)
    z2 = z1 - rowz_dstar + ((rowz_dstar + 1.0) * jnp.exp(delta) - 1.0)
    lse_r = jnp.log(z2)
    lp_reverse = delta - lse_r

    la = (th_new - th_old) + lp_reverse - lp_forward
    accept = jnp.exp(la) > au_ref[0]

    take_new = accept & is_dstar
    out_ref[0] = jnp.where(take_new, newrow, xb)


@jax.jit
def kernel(x, theta, gumbel_u, accept_u):
    th = theta.reshape(D, V)
    gu = gumbel_u.reshape(B, D, V)
    au = accept_u.reshape(B, 1, 1)
    return pl.pallas_call(
        _step_kernel,
        grid=(B,),
        in_specs=[
            pl.BlockSpec((1, D, V), lambda b: (b, 0, 0)),
            pl.BlockSpec((D, V), lambda b: (0, 0)),
            pl.BlockSpec((1, D, V), lambda b: (b, 0, 0)),
            pl.BlockSpec((1, 1, 1), lambda b: (b, 0, 0)),
        ],
        out_specs=pl.BlockSpec((1, D, V), lambda b: (b, 0, 0)),
        out_shape=jax.ShapeDtypeStruct((B, D, V), x.dtype),
    )(x, th, gu, au)


# speedup vs baseline: 5.9069x; 1.4915x over previous
import jax
import jax.numpy as jnp
from jax import lax
from jax.experimental import pallas as pl
from jax.experimental.pallas import tpu as pltpu

B, D, V = 64, 32, 8192
TEMP = 2.0
BB = 8
NBUF = 8
UNROLL = 4


def _step_kernel(x_ref, th_ref, au_ref, gu_hbm, out_ref, *scratch):
    bufs, sems = scratch[:NBUF], scratch[NBUF:]
    i = pl.program_id(0)
    th = th_ref[...]

    def gu_dma(d):
        slot = d % NBUF
        return pltpu.make_async_copy(
            gu_hbm.at[pl.ds(i * BB, BB), pl.ds(d * V, V)],
            bufs[slot],
            sems[slot],
        )

    for d in range(NBUF - UNROLL):
        gu_dma(d).start()

    iota_v2 = lax.broadcasted_iota(jnp.int32, (D, V), 1)
    c_cols, curv_cols = [], []
    for bb in range(BB):
        xb = x_ref[bb]
        c_cols.append(jnp.sum(xb * th, axis=1, keepdims=True))
        curv_cols.append(jnp.min(jnp.where(xb > 0.5, iota_v2, V),
                                 axis=1, keepdims=True))
    c2 = jnp.concatenate(c_cols, axis=1).T
    curv2 = jnp.concatenate(curv_cols, axis=1).T

    E = jnp.exp(th * (1.0 / TEMP))
    SEt = jnp.sum(E, axis=1, keepdims=True).T
    s2 = jnp.exp(c2 * (-1.0 / TEMP))

    iota_l = lax.broadcasted_iota(jnp.int32, (BB, V), 1)
    stats = []
    for base in range(0, D, UNROLL):
        for d in range(base + NBUF - UNROLL, base + NBUF):
            if d < D:
                gu_dma(d).start()
        for d in range(base, base + UNROLL):
            gu_dma(d).wait()
        for d in range(base, base + UNROLL):
            gu_d = bufs[d % NBUF][...]
            th_row = jnp.broadcast_to(th[d:d + 1, :], (BB, V))
            E_row = jnp.broadcast_to(E[d:d + 1, :], (BB, V))
            s_d = lax.slice(s2, (0, d), (BB, d + 1))
            cv_d = lax.slice(curv2, (0, d), (BB, d + 1))
            num = jnp.where(iota_l == cv_d, 0.0, E_row) * s_d
            sc = num / (-jnp.log(gu_d))
            cmax = jnp.max(sc, axis=1, keepdims=True)
            carg = jnp.min(jnp.where(sc == cmax, iota_l, V),
                           axis=1, keepdims=True)
            th_at = jnp.sum(jnp.where(iota_l == carg, th_row, 0.0),
                            axis=1, keepdims=True)
            stats.append((cmax, carg, th_at))

    gmax, vstar, th_new = stats[0]
    dstar = jnp.zeros((BB, 1), jnp.int32)
    for d in range(1, D):
        cmax, carg, th_at = stats[d]
        upd = cmax > gmax
        gmax = jnp.where(upd, cmax, gmax)
        dstar = jnp.where(upd, d, dstar)
        vstar = jnp.where(upd, carg, vstar)
        th_new = jnp.where(upd, th_at, th_new)

    iota_d1 = lax.broadcasted_iota(jnp.int32, (BB, D), 1)
    at_d = iota_d1 == dstar
    zmat = s2 * SEt - 1.0
    zsum = jnp.sum(zmat, axis=1, keepdims=True)
    rowz_at = jnp.sum(jnp.where(at_d, zmat, 0.0), axis=1, keepdims=True)
    th_old = jnp.sum(jnp.where(at_d, c2, 0.0), axis=1, keepdims=True)

    lse_f = jnp.log(zsum)
    delta = (th_old - th_new) / TEMP
    lp_forward = -delta - lse_f
    z2 = zsum - rowz_at + ((rowz_at + 1.0) * jnp.exp(delta) - 1.0)
    lp_reverse = delta - jnp.log(z2)
    la = (th_new - th_old) + lp_reverse - lp_forward
    accept = jnp.exp(la) > au_ref[...]

    iota_d2 = lax.broadcasted_iota(jnp.int32, (D, 1), 0)
    for bb in range(BB):
        acc_b = lax.slice(accept, (bb, 0), (bb + 1, 1))
        ds_b = lax.slice(dstar, (bb, 0), (bb + 1, 1))
        vs_b = lax.slice(vstar, (bb, 0), (bb + 1, 1))
        flip = (iota_d2 == ds_b) & acc_b
        row_idx = jnp.where(flip, vs_b, curv_cols[bb])
        out_ref[bb] = (iota_v2 == row_idx).astype(jnp.float32)


@jax.jit
def kernel(x, theta, gumbel_u, accept_u):
    th = theta.reshape(D, V)
    au = accept_u.reshape(B, 1)
    return pl.pallas_call(
        _step_kernel,
        grid=(B // BB,),
        in_specs=[
            pl.BlockSpec((BB, D, V), lambda i: (i, 0, 0)),
            pl.BlockSpec((D, V), lambda i: (0, 0)),
            pl.BlockSpec((BB, 1), lambda i: (i, 0)),
            pl.BlockSpec(memory_space=pl.ANY),
        ],
        out_specs=pl.BlockSpec((BB, D, V), lambda i: (i, 0, 0)),
        out_shape=jax.ShapeDtypeStruct((B, D, V), x.dtype),
        scratch_shapes=(
            [pltpu.VMEM((BB, V), jnp.float32)] * NBUF
            + [pltpu.SemaphoreType.DMA] * NBUF
        ),
    )(x, th, au, gumbel_u)


# speedup vs baseline: 6.1814x; 1.0465x over previous
import jax
import jax.numpy as jnp
from jax import lax
from jax.experimental import pallas as pl
from jax.experimental.pallas import tpu as pltpu

B, D, V = 64, 32, 8192
TEMP = 2.0
BB = 8
NBUF = 16
UNROLL = 8


def _step_kernel(x_ref, th_ref, au_ref, gu_hbm, out_ref, *scratch):
    bufs, sems = scratch[:NBUF], scratch[NBUF:]
    i = pl.program_id(0)
    th = th_ref[...]

    def gu_dma(d):
        slot = d % NBUF
        return pltpu.make_async_copy(
            gu_hbm.at[pl.ds(i * BB, BB), pl.ds(d * V, V)],
            bufs[slot],
            sems[slot],
        )

    for d in range(NBUF - UNROLL):
        gu_dma(d).start()

    iota_v2 = lax.broadcasted_iota(jnp.int32, (D, V), 1)
    c_cols, curv_cols = [], []
    for bb in range(BB):
        xb = x_ref[bb]
        c_cols.append(jnp.sum(xb * th, axis=1, keepdims=True))
        curv_cols.append(jnp.min(jnp.where(xb > 0.5, iota_v2, V),
                                 axis=1, keepdims=True))
    c2 = jnp.concatenate(c_cols, axis=1).T
    curv2 = jnp.concatenate(curv_cols, axis=1).T

    E = jnp.exp(th * (1.0 / TEMP))
    SEt = jnp.sum(E, axis=1, keepdims=True).T
    s2 = jnp.exp(c2 * (-1.0 / TEMP))

    iota_l = lax.broadcasted_iota(jnp.int32, (BB, V), 1)
    stats = []
    for base in range(0, D, UNROLL):
        for d in range(base + NBUF - UNROLL, base + NBUF):
            if d < D:
                gu_dma(d).start()
        for d in range(base, base + UNROLL):
            gu_dma(d).wait()
        for d in range(base, base + UNROLL):
            gu_d = bufs[d % NBUF][...]
            th_row = jnp.broadcast_to(th[d:d + 1, :], (BB, V))
            E_row = jnp.broadcast_to(E[d:d + 1, :], (BB, V))
            s_d = lax.slice(s2, (0, d), (BB, d + 1))
            cv_d = lax.slice(curv2, (0, d), (BB, d + 1))
            num = jnp.where(iota_l == cv_d, 0.0, E_row) * s_d
            sc = num / (-jnp.log(gu_d))
            cmax = jnp.max(sc, axis=1, keepdims=True)
            carg = jnp.min(jnp.where(sc == cmax, iota_l, V),
                           axis=1, keepdims=True)
            th_at = jnp.sum(jnp.where(iota_l == carg, th_row, 0.0),
                            axis=1, keepdims=True)
            stats.append((cmax, carg, th_at))

    gmax, vstar, th_new = stats[0]
    dstar = jnp.zeros((BB, 1), jnp.int32)
    for d in range(1, D):
        cmax, carg, th_at = stats[d]
        upd = cmax > gmax
        gmax = jnp.where(upd, cmax, gmax)
        dstar = jnp.where(upd, d, dstar)
        vstar = jnp.where(upd, carg, vstar)
        th_new = jnp.where(upd, th_at, th_new)

    iota_d1 = lax.broadcasted_iota(jnp.int32, (BB, D), 1)
    at_d = iota_d1 == dstar
    zmat = s2 * SEt - 1.0
    zsum = jnp.sum(zmat, axis=1, keepdims=True)
    rowz_at = jnp.sum(jnp.where(at_d, zmat, 0.0), axis=1, keepdims=True)
    th_old = jnp.sum(jnp.where(at_d, c2, 0.0), axis=1, keepdims=True)

    lse_f = jnp.log(zsum)
    delta = (th_old - th_new) / TEMP
    lp_forward = -delta - lse_f
    z2 = zsum - rowz_at + ((rowz_at + 1.0) * jnp.exp(delta) - 1.0)
    lp_reverse = delta - jnp.log(z2)
    la = (th_new - th_old) + lp_reverse - lp_forward
    accept = jnp.exp(la) > au_ref[...]

    iota_d2 = lax.broadcasted_iota(jnp.int32, (D, 1), 0)
    for bb in range(BB):
        acc_b = lax.slice(accept, (bb, 0), (bb + 1, 1))
        ds_b = lax.slice(dstar, (bb, 0), (bb + 1, 1))
        vs_b = lax.slice(vstar, (bb, 0), (bb + 1, 1))
        flip = (iota_d2 == ds_b) & acc_b
        row_idx = jnp.where(flip, vs_b, curv_cols[bb])
        out_ref[bb] = (iota_v2 == row_idx).astype(jnp.float32)


@jax.jit
def kernel(x, theta, gumbel_u, accept_u):
    th = theta.reshape(D, V)
    au = accept_u.reshape(B, 1)
    return pl.pallas_call(
        _step_kernel,
        grid=(B // BB,),
        in_specs=[
            pl.BlockSpec((BB, D, V), lambda i: (i, 0, 0)),
            pl.BlockSpec((D, V), lambda i: (0, 0)),
            pl.BlockSpec((BB, 1), lambda i: (i, 0)),
            pl.BlockSpec(memory_space=pl.ANY),
        ],
        out_specs=pl.BlockSpec((BB, D, V), lambda i: (i, 0, 0)),
        out_shape=jax.ShapeDtypeStruct((B, D, V), x.dtype),
        scratch_shapes=(
            [pltpu.VMEM((BB, V), jnp.float32)] * NBUF
            + [pltpu.SemaphoreType.DMA] * NBUF
        ),
    )(x, th, au, gumbel_u)


# speedup vs baseline: 6.1817x; 1.0000x over previous
import jax
import jax.numpy as jnp
from jax import lax
from jax.experimental import pallas as pl
from jax.experimental.pallas import tpu as pltpu

B, D, V = 64, 32, 8192
TEMP = 2.0
BB = 8
NBUF = 24
UNROLL = 8


def _step_kernel(x_ref, th_ref, au_ref, gu_hbm, out_ref, *scratch):
    bufs, sems = scratch[:NBUF], scratch[NBUF:]
    i = pl.program_id(0)
    th = th_ref[...]

    def gu_dma(d):
        slot = d % NBUF
        return pltpu.make_async_copy(
            gu_hbm.at[pl.ds(i * BB, BB), pl.ds(d * V, V)],
            bufs[slot],
            sems[slot],
        )

    for d in range(NBUF - UNROLL):
        gu_dma(d).start()

    iota_v2 = lax.broadcasted_iota(jnp.int32, (D, V), 1)
    c_cols, curv_cols = [], []
    for bb in range(BB):
        xb = x_ref[bb]
        c_cols.append(jnp.sum(xb * th, axis=1, keepdims=True))
        curv_cols.append(jnp.min(jnp.where(xb > 0.5, iota_v2, V),
                                 axis=1, keepdims=True))
    c2 = jnp.concatenate(c_cols, axis=1).T
    curv2 = jnp.concatenate(curv_cols, axis=1).T

    E = jnp.exp(th * (1.0 / TEMP))
    SEt = jnp.sum(E, axis=1, keepdims=True).T
    s2 = jnp.exp(c2 * (-1.0 / TEMP))

    iota_l = lax.broadcasted_iota(jnp.int32, (BB, V), 1)
    stats = []
    for base in range(0, D, UNROLL):
        for d in range(base + NBUF - UNROLL, base + NBUF):
            if d < D:
                gu_dma(d).start()
        for d in range(base, base + UNROLL):
            gu_dma(d).wait()
        for d in range(base, base + UNROLL):
            gu_d = bufs[d % NBUF][...]
            th_row = jnp.broadcast_to(th[d:d + 1, :], (BB, V))
            E_row = jnp.broadcast_to(E[d:d + 1, :], (BB, V))
            s_d = lax.slice(s2, (0, d), (BB, d + 1))
            cv_d = lax.slice(curv2, (0, d), (BB, d + 1))
            num = jnp.where(iota_l == cv_d, 0.0, E_row) * s_d
            sc = num / (-jnp.log(gu_d))
            cmax = jnp.max(sc, axis=1, keepdims=True)
            carg = jnp.min(jnp.where(sc == cmax, iota_l, V),
                           axis=1, keepdims=True)
            th_at = jnp.sum(jnp.where(iota_l == carg, th_row, 0.0),
                            axis=1, keepdims=True)
            stats.append((cmax, carg, th_at))

    gmax, vstar, th_new = stats[0]
    dstar = jnp.zeros((BB, 1), jnp.int32)
    for d in range(1, D):
        cmax, carg, th_at = stats[d]
        upd = cmax > gmax
        gmax = jnp.where(upd, cmax, gmax)
        dstar = jnp.where(upd, d, dstar)
        vstar = jnp.where(upd, carg, vstar)
        th_new = jnp.where(upd, th_at, th_new)

    iota_d1 = lax.broadcasted_iota(jnp.int32, (BB, D), 1)
    at_d = iota_d1 == dstar
    zmat = s2 * SEt - 1.0
    zsum = jnp.sum(zmat, axis=1, keepdims=True)
    rowz_at = jnp.sum(jnp.where(at_d, zmat, 0.0), axis=1, keepdims=True)
    th_old = jnp.sum(jnp.where(at_d, c2, 0.0), axis=1, keepdims=True)

    lse_f = jnp.log(zsum)
    delta = (th_old - th_new) / TEMP
    lp_forward = -delta - lse_f
    z2 = zsum - rowz_at + ((rowz_at + 1.0) * jnp.exp(delta) - 1.0)
    lp_reverse = delta - jnp.log(z2)
    la = (th_new - th_old) + lp_reverse - lp_forward
    accept = jnp.exp(la) > au_ref[...]

    iota_d2 = lax.broadcasted_iota(jnp.int32, (D, 1), 0)
    for bb in range(BB):
        acc_b = lax.slice(accept, (bb, 0), (bb + 1, 1))
        ds_b = lax.slice(dstar, (bb, 0), (bb + 1, 1))
        vs_b = lax.slice(vstar, (bb, 0), (bb + 1, 1))
        flip = (iota_d2 == ds_b) & acc_b
        row_idx = jnp.where(flip, vs_b, curv_cols[bb])
        out_ref[bb] = (iota_v2 == row_idx).astype(jnp.float32)


@jax.jit
def kernel(x, theta, gumbel_u, accept_u):
    th = theta.reshape(D, V)
    au = accept_u.reshape(B, 1)
    return pl.pallas_call(
        _step_kernel,
        grid=(B // BB,),
        in_specs=[
            pl.BlockSpec((BB, D, V), lambda i: (i, 0, 0)),
            pl.BlockSpec((D, V), lambda i: (0, 0)),
            pl.BlockSpec((BB, 1), lambda i: (i, 0)),
            pl.BlockSpec(memory_space=pl.ANY),
        ],
        out_specs=pl.BlockSpec((BB, D, V), lambda i: (i, 0, 0)),
        out_shape=jax.ShapeDtypeStruct((B, D, V), x.dtype),
        scratch_shapes=(
            [pltpu.VMEM((BB, V), jnp.float32)] * NBUF
            + [pltpu.SemaphoreType.DMA] * NBUF
        ),
    )(x, th, au, gumbel_u)


# speedup vs baseline: 6.3546x; 1.0280x over previous
import jax
import jax.numpy as jnp
from jax import lax
from jax.experimental import pallas as pl
from jax.experimental.pallas import tpu as pltpu

B, D, V = 64, 32, 8192
TEMP = 2.0
BB = 8
UNROLL = 8
NREG = D // UNROLL
NRING = 3


def _step_kernel(x_ref, th_ref, au_ref, gu_hbm, out_ref, *scratch):
    bufs, sems = scratch[:NRING], scratch[NRING:]
    i = pl.program_id(0)
    th = th_ref[...]

    def gu_dma(r):
        return pltpu.make_async_copy(
            gu_hbm.at[pl.ds(i * BB, BB), pl.ds(r * UNROLL * V, UNROLL * V)],
            bufs[r % NRING],
            sems[r % NRING],
        )

    for r in range(NRING - 1):
        gu_dma(r).start()

    iota_v2 = lax.broadcasted_iota(jnp.int32, (D, V), 1)
    c_cols, curv_cols = [], []
    for bb in range(BB):
        xb = x_ref[bb]
        c_cols.append(jnp.sum(xb * th, axis=1, keepdims=True))
        curv_cols.append(jnp.min(jnp.where(xb > 0.5, iota_v2, V),
                                 axis=1, keepdims=True))
    c2 = jnp.concatenate(c_cols, axis=1).T
    curv2 = jnp.concatenate(curv_cols, axis=1).T

    E = jnp.exp(th * (1.0 / TEMP))
    SEt = jnp.sum(E, axis=1, keepdims=True).T
    s2 = jnp.exp(c2 * (-1.0 / TEMP))

    iota_l = lax.broadcasted_iota(jnp.int32, (BB, V), 1)
    stats = []
    for r in range(NREG):
        if r + NRING - 1 < NREG:
            gu_dma(r + NRING - 1).start()
        gu_dma(r).wait()
        base = r * UNROLL
        for d in range(base, base + UNROLL):
            k = d - base
            gu_d = bufs[r % NRING][:, k * V:(k + 1) * V]
            th_row = jnp.broadcast_to(th[d:d + 1, :], (BB, V))
            E_row = jnp.broadcast_to(E[d:d + 1, :], (BB, V))
            s_d = lax.slice(s2, (0, d), (BB, d + 1))
            cv_d = lax.slice(curv2, (0, d), (BB, d + 1))
            num = jnp.where(iota_l == cv_d, 0.0, E_row) * s_d
            sc = num / (-jnp.log(gu_d))
            cmax = jnp.max(sc, axis=1, keepdims=True)
            carg = jnp.min(jnp.where(sc == cmax, iota_l, V),
                           axis=1, keepdims=True)
            th_at = jnp.sum(jnp.where(iota_l == carg, th_row, 0.0),
                            axis=1, keepdims=True)
            stats.append((cmax, carg, th_at))

    gmax, vstar, th_new = stats[0]
    dstar = jnp.zeros((BB, 1), jnp.int32)
    for d in range(1, D):
        cmax, carg, th_at = stats[d]
        upd = cmax > gmax
        gmax = jnp.where(upd, cmax, gmax)
        dstar = jnp.where(upd, d, dstar)
        vstar = jnp.where(upd, carg, vstar)
        th_new = jnp.where(upd, th_at, th_new)

    iota_d1 = lax.broadcasted_iota(jnp.int32, (BB, D), 1)
    at_d = iota_d1 == dstar
    zmat = s2 * SEt - 1.0
    zsum = jnp.sum(zmat, axis=1, keepdims=True)
    rowz_at = jnp.sum(jnp.where(at_d, zmat, 0.0), axis=1, keepdims=True)
    th_old = jnp.sum(jnp.where(at_d, c2, 0.0), axis=1, keepdims=True)

    lse_f = jnp.log(zsum)
    delta = (th_old - th_new) / TEMP
    lp_forward = -delta - lse_f
    z2 = zsum - rowz_at + ((rowz_at + 1.0) * jnp.exp(delta) - 1.0)
    lp_reverse = delta - jnp.log(z2)
    la = (th_new - th_old) + lp_reverse - lp_forward
    accept = jnp.exp(la) > au_ref[...]

    iota_d2 = lax.broadcasted_iota(jnp.int32, (D, 1), 0)
    for bb in range(BB):
        acc_b = lax.slice(accept, (bb, 0), (bb + 1, 1))
        ds_b = lax.slice(dstar, (bb, 0), (bb + 1, 1))
        vs_b = lax.slice(vstar, (bb, 0), (bb + 1, 1))
        flip = (iota_d2 == ds_b) & acc_b
        row_idx = jnp.where(flip, vs_b, curv_cols[bb])
        out_ref[bb] = (iota_v2 == row_idx).astype(jnp.float32)


@jax.jit
def kernel(x, theta, gumbel_u, accept_u):
    th = theta.reshape(D, V)
    au = accept_u.reshape(B, 1)
    return pl.pallas_call(
        _step_kernel,
        grid=(B // BB,),
        in_specs=[
            pl.BlockSpec((BB, D, V), lambda i: (i, 0, 0)),
            pl.BlockSpec((D, V), lambda i: (0, 0)),
            pl.BlockSpec((BB, 1), lambda i: (i, 0)),
            pl.BlockSpec(memory_space=pl.ANY),
        ],
        out_specs=pl.BlockSpec((BB, D, V), lambda i: (i, 0, 0)),
        out_shape=jax.ShapeDtypeStruct((B, D, V), x.dtype),
        scratch_shapes=(
            [pltpu.VMEM((BB, UNROLL * V), jnp.float32)] * NRING
            + [pltpu.SemaphoreType.DMA] * NRING
        ),
    )(x, th, au, gumbel_u)
